# baseline (speedup 1.0000x reference)
import functools

import jax
import jax.numpy as jnp
from jax import lax
from jax.experimental import pallas as pl
from jax.experimental.pallas import tpu as pltpu
from jax.experimental.pallas import tpu_sc as plsc

BATCH = 4096
SEQ = 200
DIM = 64
NUM_CORES = 2
NUM_SUBCORES = 16
NW = NUM_CORES * NUM_SUBCORES
ROWS_PER_W = BATCH // NW
G0 = 128
G1 = SEQ - G0

_mesh = plsc.VectorSubcoreMesh(core_axis_name="c", subcore_axis_name="s")


@functools.partial(
    pl.kernel,
    mesh=_mesh,
    out_type=jax.ShapeDtypeStruct((BATCH, SEQ, DIM), jnp.float32),
    scratch_types=[
        pltpu.VMEM((ROWS_PER_W, SEQ), jnp.int32),
        pltpu.VMEM((SEQ, DIM), jnp.float32),
        pltpu.VMEM((SEQ, DIM), jnp.float32),
        pltpu.SemaphoreType.DMA,
        pltpu.SemaphoreType.DMA,
        pltpu.SemaphoreType.DMA,
    ],
    compiler_params=pltpu.CompilerParams(use_tc_tiling_on_sc=False),
)
def _gather_kernel(idx_hbm, table_hbm, out_hbm, idx_v, rows0, rows1,
                   gsem, ssem0, ssem1):
    wid = lax.axis_index("s") * NUM_CORES + lax.axis_index("c")
    pltpu.sync_copy(idx_hbm.at[wid], idx_v)
    base = wid * ROWS_PER_W

    bufs = (rows0, rows1)
    ssems = (ssem0, ssem1)

    def fire_gathers(g, buf):
        pltpu.async_copy(table_hbm.at[idx_v.at[g, pl.ds(0, G0)]],
                         buf.at[pl.ds(0, G0)], gsem)
        pltpu.async_copy(table_hbm.at[idx_v.at[g, pl.ds(G0, G1)]],
                         buf.at[pl.ds(G0, G1)], gsem)

    def wait_gathers(buf):
        pltpu.make_async_copy(out_hbm.at[0], buf, gsem).wait()

    def wait_store(buf, sem):
        pltpu.make_async_copy(buf, out_hbm.at[0], sem).wait()

    fire_gathers(0, bufs[0])

    def body(i, carry):
        for b in range(2):
            g = 2 * i + b
            nb = 1 - b
            if b == 0:
                @pl.when(i >= 1)
                def _():
                    wait_store(bufs[nb], ssems[nb])
                fire_gathers(g + 1, bufs[nb])
            else:
                @pl.when(i < ROWS_PER_W // 2 - 1)
                def _():
                    wait_store(bufs[nb], ssems[nb])
                    fire_gathers(g + 1, bufs[nb])
            wait_gathers(bufs[b])
            pltpu.async_copy(bufs[b], out_hbm.at[base + g], ssems[b])
        return carry

    lax.fori_loop(0, ROWS_PER_W // 2, body, 0)

    wait_store(bufs[0], ssems[0])
    wait_store(bufs[1], ssems[1])


def kernel(inputs, weight):
    idx = inputs.astype(jnp.int32).reshape(NW, ROWS_PER_W, SEQ)
    return _gather_kernel(idx, weight)


# speedup vs baseline: 1.4289x; 1.4289x over previous
import functools

import jax
import jax.numpy as jnp
from jax import lax
from jax.experimental import pallas as pl
from jax.experimental.pallas import tpu as pltpu
from jax.experimental.pallas import tpu_sc as plsc

BATCH = 4096
SEQ = 200
DIM = 64
B_TOTAL = BATCH * SEQ
NUM_CORES = 2
NUM_SUBCORES = 16
NW = NUM_CORES * NUM_SUBCORES
ROWS_PER_W = B_TOTAL // NW // SEQ
G0 = 128
G1 = SEQ - G0

_mesh = plsc.VectorSubcoreMesh(core_axis_name="c", subcore_axis_name="s")


@functools.partial(
    pl.kernel,
    mesh=_mesh,
    out_type=jax.ShapeDtypeStruct((B_TOTAL, 2 * DIM), jnp.float32),
    scratch_types=[
        pltpu.VMEM((ROWS_PER_W, SEQ), jnp.int32),
        pltpu.VMEM((SEQ, DIM), jnp.float32),
        pltpu.VMEM((SEQ, DIM), jnp.float32),
        pltpu.SemaphoreType.DMA,
        pltpu.SemaphoreType.DMA,
        pltpu.SemaphoreType.DMA,
    ],
    compiler_params=pltpu.CompilerParams(use_tc_tiling_on_sc=False),
)
def _gather_kernel(idx_hbm, table_hbm, out_hbm, idx_v, rows0, rows1,
                   gsem, ssem0, ssem1):
    wid = lax.axis_index("s") * NUM_CORES + lax.axis_index("c")
    pltpu.sync_copy(idx_hbm.at[wid], idx_v)
    base = wid * ROWS_PER_W

    bufs = (rows0, rows1)
    ssems = (ssem0, ssem1)

    def fire_gathers(g, buf):
        pltpu.async_copy(table_hbm.at[idx_v.at[g, pl.ds(0, G0)]],
                         buf.at[pl.ds(0, G0)], gsem)
        pltpu.async_copy(table_hbm.at[idx_v.at[g, pl.ds(G0, G1)]],
                         buf.at[pl.ds(G0, G1)], gsem)

    def wait_gathers(buf):
        pltpu.make_async_copy(out_hbm.at[pl.ds(0, SEQ), pl.ds(0, DIM)],
                              buf, gsem).wait()

    def store(g, buf, sem):
        pltpu.async_copy(
            buf, out_hbm.at[pl.ds((base + g) * SEQ, SEQ), pl.ds(0, DIM)], sem)

    def wait_store(buf, sem):
        pltpu.make_async_copy(buf, out_hbm.at[pl.ds(0, SEQ), pl.ds(0, DIM)],
                              sem).wait()

    fire_gathers(0, bufs[0])

    def body(i, carry):
        for b in range(2):
            g = 2 * i + b
            nb = 1 - b
            if b == 0:
                @pl.when(i >= 1)
                def _():
                    wait_store(bufs[nb], ssems[nb])
                fire_gathers(g + 1, bufs[nb])
            else:
                @pl.when(i < ROWS_PER_W // 2 - 1)
                def _():
                    wait_store(bufs[nb], ssems[nb])
                    fire_gathers(g + 1, bufs[nb])
            wait_gathers(bufs[b])
            store(g, bufs[b], ssems[b])
        return carry

    lax.fori_loop(0, ROWS_PER_W // 2, body, 0)

    wait_store(bufs[0], ssems[0])
    wait_store(bufs[1], ssems[1])


def kernel(inputs, weight):
    w2 = jnp.pad(weight, ((0, 0), (0, DIM))).reshape(2 * weight.shape[0], DIM)
    idx2 = (inputs.astype(jnp.int32) * 2).reshape(NW, ROWS_PER_W, SEQ)
    outp = _gather_kernel(idx2, w2)
    return outp[:, :DIM].reshape(BATCH, SEQ, DIM)
